# baseline (speedup 1.0000x reference)
import functools

import jax
import jax.numpy as jnp
from jax import lax
from jax.experimental import pallas as pl
from jax.experimental.pallas import tpu as pltpu
from jax.experimental.pallas import tpu_sc as plsc

_N = 10000
_E = 640000
_D = 128
_B = 64
_H = 64
_OUT = 128

_NC = 2
_NS = 16
_NW = _NC * _NS
_EPW = _E // _NW
_CH = 128
_NCHUNK = _EPW // _CH
_TAIL = _EPW - _NCHUNK * _CH
_HH = _H // 2
_SBC = 32
_NSB = 10
_CPT = _SBC * _NSB
_ER = _NS * _CPT
_RBASE = 624
_RP = 16

_mesh = plsc.VectorSubcoreMesh(core_axis_name="c", subcore_axis_name="s")


_DCPT = _ER // _NW


def _deg_body(dst_hbm, ew_hbm, out_hbm, dstb, ewb,
              buf0, buf1, buf2, buf3, zbuf, shacc, s0, s1, s2, s3):
    cid = lax.axis_index("c")
    sid = lax.axis_index("s")
    wid = cid * _NS + sid
    bufs = (buf0, buf1, buf2, buf3)
    ssem = (s0, s1, s2, s3)

    pltpu.sync_copy(dst_hbm.at[pl.ds(wid * _DCPT, _DCPT)], dstb)
    pltpu.sync_copy(ew_hbm.at[pl.ds(wid * _DCPT, _DCPT)], ewb)

    for k in range(_RP):
        zbuf[k, :] = jnp.zeros((16,), jnp.float32)
    npz = jnp.where(sid == _NS - 1, 40, 39)

    def _zpiece(j, carry):
        pltpu.sync_copy(zbuf, shacc.at[pl.ds(sid * _RBASE + j * _RP, _RP)])
        return carry
    lax.fori_loop(0, npz, _zpiece, 0)
    plsc.subcore_barrier()

    def _step(i, carry):
        for b in range(4):
            t = i * 4 + b

            @pl.when(t >= 4)
            def _():
                pltpu.make_async_copy(
                    bufs[b], shacc.at[dstb.at[t - 4]], ssem[b]).wait()

            def _fill(g, c2):
                wv = ewb[t, pl.ds(g * 16, 16)]
                for e in range(16):
                    bufs[b][g * 16 + e, :] = jnp.broadcast_to(wv[e], (16,))
                return c2
            lax.fori_loop(0, _CH // 16, _fill, 0)
            pltpu.async_copy(bufs[b], shacc.at[dstb.at[t]], ssem[b], add=True)
        return carry
    lax.fori_loop(0, _DCPT // 4, _step, 0)

    for b in range(4):
        t_last = _DCPT - 4 + b
        pltpu.make_async_copy(
            bufs[b], shacc.at[dstb.at[t_last]], ssem[b]).wait()

    plsc.subcore_barrier()

    def _cpiece(j, carry):
        r0 = sid * _RBASE + j * _RP
        pltpu.sync_copy(shacc.at[pl.ds(r0, _RP)], zbuf)
        pltpu.sync_copy(zbuf, out_hbm.at[cid, pl.ds(r0, _RP)])
        return carry
    lax.fori_loop(0, npz, _cpiece, 0)


_deg_call = pl.kernel(
    _deg_body,
    out_type=jax.ShapeDtypeStruct((_NC, _N, 16), jnp.float32),
    mesh=_mesh,
    compiler_params=pltpu.CompilerParams(use_tc_tiling_on_sc=False),
    scratch_types=[
        pltpu.VMEM((_DCPT, _CH), jnp.int32),
        pltpu.VMEM((_DCPT, _CH), jnp.float32),
        pltpu.VMEM((_CH, 16), jnp.float32),
        pltpu.VMEM((_CH, 16), jnp.float32),
        pltpu.VMEM((_CH, 16), jnp.float32),
        pltpu.VMEM((_CH, 16), jnp.float32),
        pltpu.VMEM((_RP, 16), jnp.float32),
        pltpu.VMEM_SHARED((_N, 16), jnp.float32),
        pltpu.SemaphoreType.DMA,
        pltpu.SemaphoreType.DMA,
        pltpu.SemaphoreType.DMA,
        pltpu.SemaphoreType.DMA,
    ],
)


def _prop_body(u_hbm, src_hbm, dst_hbm, ew_hbm, out_hbm,
               srcb0, dstb0, ewb0, srcb1, dstb1, ewb1,
               buf0, buf1, buf2, buf3, zbuf, shacc, ushr,
               g0, g1, g2, g3, s0, s1, s2, s3, p0, p1):
    cid = lax.axis_index("c")
    sid = lax.axis_index("s")
    bufs = (buf0, buf1, buf2, buf3)
    gsem = (g0, g1, g2, g3)
    ssem = (s0, s1, s2, s3)
    idxs = ((srcb0, dstb0, ewb0), (srcb1, dstb1, ewb1))
    psem = (p0, p1)

    r0u = sid * (_N // _NS)
    pltpu.sync_copy(u_hbm.at[cid, pl.ds(r0u, _N // _NS)],
                    ushr.at[pl.ds(r0u, _N // _NS)])

    for k in range(_RP):
        for c in range(2):
            zbuf[k, pl.ds(c * 16, 16)] = jnp.zeros((16,), jnp.float32)
    npz = jnp.where(sid == _NS - 1, 40, 39)

    def _zpiece(j, carry):
        pltpu.sync_copy(zbuf, shacc.at[pl.ds(sid * _RBASE + j * _RP, _RP)])
        return carry
    lax.fori_loop(0, npz, _zpiece, 0)
    plsc.subcore_barrier()

    def _sb_load(n, p):
        base = sid * _CPT + n * _SBC
        pltpu.async_copy(src_hbm.at[pl.ds(base, _SBC)], idxs[p][0], psem[p])
        pltpu.async_copy(dst_hbm.at[pl.ds(base, _SBC)], idxs[p][1], psem[p])
        pltpu.async_copy(ew_hbm.at[pl.ds(base, _SBC)], idxs[p][2], psem[p])

    def _sb_wait(p):
        pltpu.make_async_copy(
            src_hbm.at[pl.ds(0, _SBC)], idxs[p][0], psem[p]).wait()
        pltpu.make_async_copy(
            dst_hbm.at[pl.ds(0, _SBC)], idxs[p][1], psem[p]).wait()
        pltpu.make_async_copy(
            ew_hbm.at[pl.ds(0, _SBC)], idxs[p][2], psem[p]).wait()

    _sb_load(0, 0)
    _sb_load(1, 1)

    def _outer(i, carry):
        for p in range(2):
            n = i * 2 + p
            sbs, dbs, ebs = idxs[p]
            _sb_wait(p)

            def _gath(t, b):
                for h in range(2):
                    pltpu.async_copy(ushr.at[sbs.at[2 * t + h]],
                                     bufs[b].at[pl.ds(h * _CH, _CH)],
                                     gsem[b])

            def _gath_wait(t, b):
                for h in range(2):
                    pltpu.make_async_copy(
                        ushr.at[sbs.at[2 * t + h]],
                        bufs[b].at[pl.ds(h * _CH, _CH)], gsem[b]).wait()

            def _scat(t, b, add):
                for h in range(2):
                    pltpu.async_copy(bufs[b].at[pl.ds(h * _CH, _CH)],
                                     shacc.at[dbs.at[2 * t + h]], ssem[b],
                                     add=add)

            def _scat_wait(t, b):
                for h in range(2):
                    pltpu.make_async_copy(
                        bufs[b].at[pl.ds(h * _CH, _CH)],
                        shacc.at[dbs.at[2 * t + h]], ssem[b]).wait()

            for t in range(3):
                _gath(t, t)

            nstep = _SBC // 2

            def _inner(q, c2):
                for b in range(4):
                    t = q * 4 + b
                    bn = (b + 2) % 4

                    @pl.when(jnp.logical_and(t >= 2, t + 2 < nstep))
                    def _():
                        _scat_wait(t - 2, bn)

                    @pl.when(jnp.logical_and(t >= 1, t + 2 < nstep))
                    def _():
                        _gath(t + 2, bn)

                    _gath_wait(t, b)

                    def _grp(g, c3):
                        wv = ebs[2 * t + g // 8, pl.ds((g % 8) * 16, 16)]
                        for e in range(16):
                            k = g * 16 + e
                            w = wv[e]
                            for c in range(2):
                                bufs[b][k, pl.ds(c * 16, 16)] = (
                                    bufs[b][k, pl.ds(c * 16, 16)] * w)
                        return c3
                    lax.fori_loop(0, 2 * (_CH // 16), _grp, 0)
                    _scat(t, b, True)
                return c2
            lax.fori_loop(0, nstep // 4, _inner, 0)

            for b in range(4):
                _scat_wait(nstep - 4 + b, b)

            @pl.when(n + 2 < _NSB)
            def _():
                _sb_load(n + 2, p)
        return carry
    lax.fori_loop(0, _NSB // 2, _outer, 0)

    plsc.subcore_barrier()

    def _cpiece(j, carry):
        r0 = sid * _RBASE + j * _RP
        pltpu.sync_copy(shacc.at[pl.ds(r0, _RP)], zbuf)
        pltpu.sync_copy(zbuf, out_hbm.at[cid, pl.ds(r0, _RP)])
        return carry
    lax.fori_loop(0, npz, _cpiece, 0)


_prop_call = pl.kernel(
    _prop_body,
    out_type=jax.ShapeDtypeStruct((_NC, _N, _HH), jnp.float32),
    mesh=_mesh,
    compiler_params=pltpu.CompilerParams(use_tc_tiling_on_sc=False),
    scratch_types=[
        pltpu.VMEM((_SBC, _CH), jnp.int32),
        pltpu.VMEM((_SBC, _CH), jnp.int32),
        pltpu.VMEM((_SBC, _CH), jnp.float32),
        pltpu.VMEM((_SBC, _CH), jnp.int32),
        pltpu.VMEM((_SBC, _CH), jnp.int32),
        pltpu.VMEM((_SBC, _CH), jnp.float32),
        pltpu.VMEM((2 * _CH, _HH), jnp.float32),
        pltpu.VMEM((2 * _CH, _HH), jnp.float32),
        pltpu.VMEM((2 * _CH, _HH), jnp.float32),
        pltpu.VMEM((2 * _CH, _HH), jnp.float32),
        pltpu.VMEM((_RP, _HH), jnp.float32),
        pltpu.VMEM_SHARED((_N, _HH), jnp.float32),
        pltpu.VMEM_SHARED((_N, _HH), jnp.float32),
        pltpu.SemaphoreType.DMA,
        pltpu.SemaphoreType.DMA,
        pltpu.SemaphoreType.DMA,
        pltpu.SemaphoreType.DMA,
        pltpu.SemaphoreType.DMA,
        pltpu.SemaphoreType.DMA,
        pltpu.SemaphoreType.DMA,
        pltpu.SemaphoreType.DMA,
        pltpu.SemaphoreType.DMA,
        pltpu.SemaphoreType.DMA,
    ],
)


_BLK = 1000
_NBLK = _N // _BLK


def _tc_pre(x, dp0, dp1, atom_emb, w1a, w1b):
    def body(x_ref, d0_ref, d1_ref, ae_ref, wa_ref, wb_ref, out_ref):
        xv = x_ref[...]
        dis = lax.rsqrt(d0_ref[:, 0] + d1_ref[:, 0] + 1.0)
        ids = xv[:, 0].astype(jnp.int32)
        oh = (ids[:, None] == lax.broadcasted_iota(jnp.int32, (1, 20), 1)
              ).astype(jnp.float32)
        embw = jnp.dot(ae_ref[...], wb_ref[...],
                       preferred_element_type=jnp.float32)
        xw = (jnp.dot(xv, wa_ref[...], preferred_element_type=jnp.float32)
              + jnp.dot(oh, embw, preferred_element_type=jnp.float32))
        v = dis[:, None] * xw
        out_ref[0, :, :] = v[:, :_HH]
        out_ref[1, :, :] = v[:, _HH:]

    return pl.pallas_call(
        body,
        grid=(_NBLK,),
        in_specs=[
            pl.BlockSpec((_BLK, _D), lambda i: (i, 0)),
            pl.BlockSpec((_BLK, 16), lambda i: (i, 0)),
            pl.BlockSpec((_BLK, 16), lambda i: (i, 0)),
            pl.BlockSpec((20, 32), lambda i: (0, 0)),
            pl.BlockSpec((_D, _H), lambda i: (0, 0)),
            pl.BlockSpec((32, _H), lambda i: (0, 0)),
        ],
        out_specs=pl.BlockSpec((_NC, _BLK, _HH), lambda i: (0, i, 0)),
        out_shape=jax.ShapeDtypeStruct((_NC, _N, _HH), jnp.float32),
    )(x, dp0, dp1, atom_emb, w1a, w1b)


def _tc_mid(sp, u, dp0, dp1, b2d, wn):
    def body(sp_ref, u_ref, d0_ref, d1_ref, b_ref, w_ref, out_ref):
        dis = lax.rsqrt(d0_ref[:, 0] + d1_ref[:, 0] + 1.0)
        s = jnp.concatenate([sp_ref[0], sp_ref[1]], axis=1)
        uu = jnp.concatenate([u_ref[0], u_ref[1]], axis=1)
        h = jnp.maximum(dis[:, None] * (s + uu) + b_ref[0, :], 0.0)
        v = dis[:, None] * jnp.dot(
            h, w_ref[...], preferred_element_type=jnp.float32)
        out_ref[0, :, :] = v[:, :_HH]
        out_ref[1, :, :] = v[:, _HH:]

    return pl.pallas_call(
        body,
        grid=(_NBLK,),
        in_specs=[
            pl.BlockSpec((_NC, _BLK, _HH), lambda i: (0, i, 0)),
            pl.BlockSpec((_NC, _BLK, _HH), lambda i: (0, i, 0)),
            pl.BlockSpec((_BLK, 16), lambda i: (i, 0)),
            pl.BlockSpec((_BLK, 16), lambda i: (i, 0)),
            pl.BlockSpec((1, _H), lambda i: (0, 0)),
            pl.BlockSpec((_H, _H), lambda i: (0, 0)),
        ],
        out_specs=pl.BlockSpec((_NC, _BLK, _HH), lambda i: (0, i, 0)),
        out_shape=jax.ShapeDtypeStruct((_NC, _N, _HH), jnp.float32),
    )(sp, u, dp0, dp1, b2d, wn)


def _tc_final(sp, u, dp0, dp1, b2d, wl, bl2d, batch3, prot2d, prot_emb,
              a1w, a1b, a2w, a2b, a3w, a3b, a4w, a4b):
    def body(sp_ref, u_ref, d0_ref, d1_ref, b_ref, wl_ref, bl_ref,
             bt_ref, pr_ref, pe_ref, A1_ref, c1_ref, A2_ref, c2_ref, A3_ref,
             c3_ref, A4_ref, c4_ref, out_ref, acc_s, acc_c):
        i = pl.program_id(0)
        dis = lax.rsqrt(d0_ref[:, 0] + d1_ref[:, 0] + 1.0)
        s = jnp.concatenate([sp_ref[0], sp_ref[1]], axis=1)
        uu = jnp.concatenate([u_ref[0], u_ref[1]], axis=1)
        h = jnp.maximum(dis[:, None] * (s + uu) + b_ref[0, :], 0.0)
        y = jnp.dot(h, wl_ref[...], preferred_element_type=jnp.float32) \
            + bl_ref[0, :]
        bb = bt_ref[0, 0, :]
        oh = (lax.broadcasted_iota(jnp.int32, (_B, 1), 0) == bb[None, :]
              ).astype(jnp.float32)
        ps = jnp.dot(oh, y, preferred_element_type=jnp.float32)
        pc = jnp.sum(oh, axis=1, keepdims=True)

        @pl.when(i == 0)
        def _():
            acc_s[...] = ps
            acc_c[...] = jnp.broadcast_to(pc, (_B, _OUT))

        @pl.when(i > 0)
        def _():
            acc_s[...] += ps
            acc_c[...] += jnp.broadcast_to(pc, (_B, _OUT))

        @pl.when(i == _NBLK - 1)
        def _():
            g = acc_s[...] / acc_c[...]
            pr = pr_ref[0, :]
            oh3 = (pr[:, None] == lax.broadcasted_iota(jnp.int32, (1, 3), 1)
                   ).astype(jnp.float32)
            pe = jnp.maximum(
                jnp.dot(oh3, pe_ref[...], preferred_element_type=jnp.float32),
                0.0)
            z = jnp.concatenate(
                [g, pe, jnp.zeros((_B, 6), jnp.float32)], axis=1)
            z = jnp.maximum(
                jnp.dot(z, A1_ref[...], preferred_element_type=jnp.float32)
                + c1_ref[0, :], 0.0)
            z = jnp.maximum(
                jnp.dot(z, A2_ref[...], preferred_element_type=jnp.float32)
                + c2_ref[0, :], 0.0)
            z = jnp.maximum(
                jnp.dot(z, A3_ref[...], preferred_element_type=jnp.float32)
                + c3_ref[0, :], 0.0)
            out_ref[...] = jax.nn.sigmoid(
                jnp.dot(z, A4_ref[...], preferred_element_type=jnp.float32)
                + c4_ref[0, :])

    return pl.pallas_call(
        body,
        grid=(_NBLK,),
        in_specs=[
            pl.BlockSpec((_NC, _BLK, _HH), lambda i: (0, i, 0)),
            pl.BlockSpec((_NC, _BLK, _HH), lambda i: (0, i, 0)),
            pl.BlockSpec((_BLK, 16), lambda i: (i, 0)),
            pl.BlockSpec((_BLK, 16), lambda i: (i, 0)),
            pl.BlockSpec((1, _H), lambda i: (0, 0)),
            pl.BlockSpec((_H, _OUT), lambda i: (0, 0)),
            pl.BlockSpec((1, _OUT), lambda i: (0, 0)),
            pl.BlockSpec((1, 1, _BLK), lambda i: (i, 0, 0)),
            pl.BlockSpec((1, _B), lambda i: (0, 0)),
            pl.BlockSpec((3, 10), lambda i: (0, 0)),
            pl.BlockSpec((144, 128), lambda i: (0, 0)),
            pl.BlockSpec((1, 128), lambda i: (0, 0)),
            pl.BlockSpec((128, 96), lambda i: (0, 0)),
            pl.BlockSpec((1, 96), lambda i: (0, 0)),
            pl.BlockSpec((96, 32), lambda i: (0, 0)),
            pl.BlockSpec((1, 32), lambda i: (0, 0)),
            pl.BlockSpec((32, 1), lambda i: (0, 0)),
            pl.BlockSpec((1, 1), lambda i: (0, 0)),
        ],
        out_specs=pl.BlockSpec((_B, 1), lambda i: (0, 0)),
        out_shape=jax.ShapeDtypeStruct((_B, 1), jnp.float32),
        scratch_shapes=[
            pltpu.VMEM((_B, _OUT), jnp.float32),
            pltpu.VMEM((_B, _OUT), jnp.float32),
        ],
    )(sp, u, dp0, dp1, b2d, wl, bl2d, batch3, prot2d, prot_emb,
      a1w, a1b, a2w, a2b, a3w, a3b, a4w, a4b)


def kernel(x, edge_index, edge_attr, batch, protein, atom_emb, prot_emb,
           W1, b1, W2, b2, W3, b3, W4, b4, Wl, bl, A1, a1, A2, a2, A3, a3,
           A4, a4):
    src = edge_index[0].astype(jnp.int32)
    dst = edge_index[1].astype(jnp.int32)
    ew = edge_attr

    npad = _ER * _CH - _E
    src2 = jnp.pad(src, (0, npad)).reshape(_ER, _CH)
    dst2 = jnp.pad(dst, (0, npad)).reshape(_ER, _CH)
    ew2 = jnp.pad(ew, (0, npad)).reshape(_ER, _CH)

    deg_part = _deg_call(dst2, ew2)
    dp0 = deg_part[0]
    dp1 = deg_part[1]

    u = _tc_pre(x, dp0, dp1, atom_emb, W1[:_D], W1[_D:])

    for (b_cur, w_next) in ((b1, W2), (b2, W3), (b3, W4)):
        sp = _prop_call(u, src2, dst2, ew2)
        u = _tc_mid(sp, u, dp0, dp1, b_cur.reshape(1, _H), w_next)

    sp = _prop_call(u, src2, dst2, ew2)
    out = _tc_final(
        sp, u, dp0, dp1, b4.reshape(1, _H), Wl, bl.reshape(1, _OUT),
        batch.astype(jnp.int32).reshape(_NBLK, 1, _BLK),
        protein.astype(jnp.int32).reshape(1, _B), prot_emb,
        jnp.pad(A1, ((0, 6), (0, 0))), a1.reshape(1, 128),
        A2, a2.reshape(1, 96), A3, a3.reshape(1, 32),
        A4, a4.reshape(1, 1))
    return out


# speedup vs baseline: 1.0734x; 1.0734x over previous
import functools

import jax
import jax.numpy as jnp
from jax import lax
from jax.experimental import pallas as pl
from jax.experimental.pallas import tpu as pltpu
from jax.experimental.pallas import tpu_sc as plsc

_N = 10000
_E = 640000
_D = 128
_B = 64
_H = 64
_OUT = 128

_NC = 2
_NS = 16
_NW = _NC * _NS
_EPW = _E // _NW
_CH = 128
_NCHUNK = _EPW // _CH
_TAIL = _EPW - _NCHUNK * _CH
_HH = _H // 2
_SBC = 32
_NSB = 10
_CPT = _SBC * _NSB
_ER = _NS * _CPT
_RBASE = 624
_RP = 16

_mesh = plsc.VectorSubcoreMesh(core_axis_name="c", subcore_axis_name="s")


_DCPT = _ER // _NW


def _deg_body(dst_hbm, ew_hbm, out_hbm, dstb, ewb,
              buf0, buf1, buf2, buf3, zbuf, shacc, s0, s1, s2, s3):
    cid = lax.axis_index("c")
    sid = lax.axis_index("s")
    wid = cid * _NS + sid
    bufs = (buf0, buf1, buf2, buf3)
    ssem = (s0, s1, s2, s3)

    pltpu.sync_copy(dst_hbm.at[pl.ds(wid * _DCPT, _DCPT)], dstb)
    pltpu.sync_copy(ew_hbm.at[pl.ds(wid * _DCPT, _DCPT)], ewb)

    for k in range(_RP):
        zbuf[k, :] = jnp.zeros((16,), jnp.float32)
    npz = jnp.where(sid == _NS - 1, 40, 39)

    def _zpiece(j, carry):
        pltpu.sync_copy(zbuf, shacc.at[pl.ds(sid * _RBASE + j * _RP, _RP)])
        return carry
    lax.fori_loop(0, npz, _zpiece, 0)
    plsc.subcore_barrier()

    def _step(i, carry):
        for b in range(4):
            t = i * 4 + b

            @pl.when(t >= 4)
            def _():
                pltpu.make_async_copy(
                    bufs[b], shacc.at[dstb.at[t - 4]], ssem[b]).wait()

            def _fill(g, c2):
                wv = ewb[t, pl.ds(g * 16, 16)]
                for e in range(16):
                    bufs[b][g * 16 + e, :] = jnp.broadcast_to(wv[e], (16,))
                return c2
            lax.fori_loop(0, _CH // 16, _fill, 0)
            pltpu.async_copy(bufs[b], shacc.at[dstb.at[t]], ssem[b], add=True)
        return carry
    lax.fori_loop(0, _DCPT // 4, _step, 0)

    for b in range(4):
        t_last = _DCPT - 4 + b
        pltpu.make_async_copy(
            bufs[b], shacc.at[dstb.at[t_last]], ssem[b]).wait()

    plsc.subcore_barrier()

    def _cpiece(j, carry):
        r0 = sid * _RBASE + j * _RP
        pltpu.sync_copy(shacc.at[pl.ds(r0, _RP)], zbuf)
        pltpu.sync_copy(zbuf, out_hbm.at[cid, pl.ds(r0, _RP)])
        return carry
    lax.fori_loop(0, npz, _cpiece, 0)


_deg_call = pl.kernel(
    _deg_body,
    out_type=jax.ShapeDtypeStruct((_NC, _N, 16), jnp.float32),
    mesh=_mesh,
    compiler_params=pltpu.CompilerParams(use_tc_tiling_on_sc=False),
    scratch_types=[
        pltpu.VMEM((_DCPT, _CH), jnp.int32),
        pltpu.VMEM((_DCPT, _CH), jnp.float32),
        pltpu.VMEM((_CH, 16), jnp.float32),
        pltpu.VMEM((_CH, 16), jnp.float32),
        pltpu.VMEM((_CH, 16), jnp.float32),
        pltpu.VMEM((_CH, 16), jnp.float32),
        pltpu.VMEM((_RP, 16), jnp.float32),
        pltpu.VMEM_SHARED((_N, 16), jnp.float32),
        pltpu.SemaphoreType.DMA,
        pltpu.SemaphoreType.DMA,
        pltpu.SemaphoreType.DMA,
        pltpu.SemaphoreType.DMA,
    ],
)


def _prop_body(u_hbm, src_hbm, dst_hbm, ew_hbm, out_hbm,
               srcb0, dstb0, ewb0, srcb1, dstb1, ewb1,
               buf0, buf1, buf2, buf3, zbuf, shacc, ushr,
               g0, g1, g2, g3, s0, s1, s2, s3, p0, p1):
    cid = lax.axis_index("c")
    sid = lax.axis_index("s")
    bufs = (buf0, buf1, buf2, buf3)
    gsem = (g0, g1, g2, g3)
    ssem = (s0, s1, s2, s3)
    idxs = ((srcb0, dstb0, ewb0), (srcb1, dstb1, ewb1))
    psem = (p0, p1)

    r0u = sid * (_N // _NS)
    pltpu.sync_copy(u_hbm.at[cid, pl.ds(r0u, _N // _NS)],
                    ushr.at[pl.ds(r0u, _N // _NS)])

    for k in range(_RP):
        for c in range(2):
            zbuf[k, pl.ds(c * 16, 16)] = jnp.zeros((16,), jnp.float32)
    npz = jnp.where(sid == _NS - 1, 40, 39)

    def _zpiece(j, carry):
        pltpu.sync_copy(zbuf, shacc.at[pl.ds(sid * _RBASE + j * _RP, _RP)])
        return carry
    lax.fori_loop(0, npz, _zpiece, 0)
    plsc.subcore_barrier()

    def _sb_load(n, p):
        base = sid * _CPT + n * _SBC
        pltpu.async_copy(src_hbm.at[pl.ds(base, _SBC)], idxs[p][0], psem[p])
        pltpu.async_copy(dst_hbm.at[pl.ds(base, _SBC)], idxs[p][1], psem[p])
        pltpu.async_copy(ew_hbm.at[pl.ds(base, _SBC)], idxs[p][2], psem[p])

    def _sb_wait(p):
        pltpu.make_async_copy(
            src_hbm.at[pl.ds(0, _SBC)], idxs[p][0], psem[p]).wait()
        pltpu.make_async_copy(
            dst_hbm.at[pl.ds(0, _SBC)], idxs[p][1], psem[p]).wait()
        pltpu.make_async_copy(
            ew_hbm.at[pl.ds(0, _SBC)], idxs[p][2], psem[p]).wait()

    _sb_load(0, 0)
    _sb_load(1, 1)

    def _outer(i, carry):
        for p in range(2):
            n = i * 2 + p
            sbs, dbs, ebs = idxs[p]
            _sb_wait(p)
            for t in range(3):
                pltpu.async_copy(ushr.at[sbs.at[t]], bufs[t], gsem[t])

            def _inner(q, c2):
                for b in range(4):
                    t = q * 4 + b
                    bn = (b + 2) % 4

                    @pl.when(jnp.logical_and(t >= 2, t + 2 < _SBC))
                    def _():
                        pltpu.make_async_copy(
                            bufs[bn], shacc.at[dbs.at[t - 2]],
                            ssem[bn]).wait()

                    @pl.when(jnp.logical_and(t >= 1, t + 2 < _SBC))
                    def _():
                        pltpu.async_copy(
                            ushr.at[sbs.at[t + 2]], bufs[bn], gsem[bn])

                    pltpu.make_async_copy(
                        ushr.at[sbs.at[t]], bufs[b], gsem[b]).wait()

                    def _grp(g, c3):
                        wv = ebs[t, pl.ds(g * 16, 16)]
                        for e in range(16):
                            k = g * 16 + e
                            w = wv[e]
                            for c in range(2):
                                bufs[b][k, pl.ds(c * 16, 16)] = (
                                    bufs[b][k, pl.ds(c * 16, 16)] * w)
                        return c3
                    lax.fori_loop(0, _CH // 16, _grp, 0)
                    pltpu.async_copy(
                        bufs[b], shacc.at[dbs.at[t]], ssem[b], add=True)
                return c2
            lax.fori_loop(0, _SBC // 4, _inner, 0)

            for b in range(4):
                t_last = _SBC - 4 + b
                pltpu.make_async_copy(
                    bufs[b], shacc.at[dbs.at[t_last]], ssem[b]).wait()

            @pl.when(n + 2 < _NSB)
            def _():
                _sb_load(n + 2, p)
        return carry
    lax.fori_loop(0, _NSB // 2, _outer, 0)

    plsc.subcore_barrier()

    def _cpiece(j, carry):
        r0 = sid * _RBASE + j * _RP
        pltpu.sync_copy(shacc.at[pl.ds(r0, _RP)], zbuf)
        pltpu.sync_copy(zbuf, out_hbm.at[cid, pl.ds(r0, _RP)])
        return carry
    lax.fori_loop(0, npz, _cpiece, 0)


_prop_call = pl.kernel(
    _prop_body,
    out_type=jax.ShapeDtypeStruct((_NC, _N, _HH), jnp.float32),
    mesh=_mesh,
    compiler_params=pltpu.CompilerParams(use_tc_tiling_on_sc=False),
    scratch_types=[
        pltpu.VMEM((_SBC, _CH), jnp.int32),
        pltpu.VMEM((_SBC, _CH), jnp.int32),
        pltpu.VMEM((_SBC, _CH), jnp.float32),
        pltpu.VMEM((_SBC, _CH), jnp.int32),
        pltpu.VMEM((_SBC, _CH), jnp.int32),
        pltpu.VMEM((_SBC, _CH), jnp.float32),
        pltpu.VMEM((_CH, _HH), jnp.float32),
        pltpu.VMEM((_CH, _HH), jnp.float32),
        pltpu.VMEM((_CH, _HH), jnp.float32),
        pltpu.VMEM((_CH, _HH), jnp.float32),
        pltpu.VMEM((_RP, _HH), jnp.float32),
        pltpu.VMEM_SHARED((_N, _HH), jnp.float32),
        pltpu.VMEM_SHARED((_N, _HH), jnp.float32),
        pltpu.SemaphoreType.DMA,
        pltpu.SemaphoreType.DMA,
        pltpu.SemaphoreType.DMA,
        pltpu.SemaphoreType.DMA,
        pltpu.SemaphoreType.DMA,
        pltpu.SemaphoreType.DMA,
        pltpu.SemaphoreType.DMA,
        pltpu.SemaphoreType.DMA,
        pltpu.SemaphoreType.DMA,
        pltpu.SemaphoreType.DMA,
    ],
)


_BLK = 1000
_NBLK = _N // _BLK


def _tc_pre(x, dp0, dp1, atom_emb, w1a, w1b):
    def body(x_ref, d0_ref, d1_ref, ae_ref, wa_ref, wb_ref, out_ref):
        xv = x_ref[...]
        dis = lax.rsqrt(d0_ref[:, 0] + d1_ref[:, 0] + 1.0)
        ids = xv[:, 0].astype(jnp.int32)
        oh = (ids[:, None] == lax.broadcasted_iota(jnp.int32, (1, 20), 1)
              ).astype(jnp.float32)
        embw = jnp.dot(ae_ref[...], wb_ref[...],
                       preferred_element_type=jnp.float32)
        xw = (jnp.dot(xv, wa_ref[...], preferred_element_type=jnp.float32)
              + jnp.dot(oh, embw, preferred_element_type=jnp.float32))
        v = dis[:, None] * xw
        out_ref[0, :, :] = v[:, :_HH]
        out_ref[1, :, :] = v[:, _HH:]

    return pl.pallas_call(
        body,
        grid=(_NBLK,),
        in_specs=[
            pl.BlockSpec((_BLK, _D), lambda i: (i, 0)),
            pl.BlockSpec((_BLK, 16), lambda i: (i, 0)),
            pl.BlockSpec((_BLK, 16), lambda i: (i, 0)),
            pl.BlockSpec((20, 32), lambda i: (0, 0)),
            pl.BlockSpec((_D, _H), lambda i: (0, 0)),
            pl.BlockSpec((32, _H), lambda i: (0, 0)),
        ],
        out_specs=pl.BlockSpec((_NC, _BLK, _HH), lambda i: (0, i, 0)),
        out_shape=jax.ShapeDtypeStruct((_NC, _N, _HH), jnp.float32),
    )(x, dp0, dp1, atom_emb, w1a, w1b)


def _tc_mid(sp, u, dp0, dp1, b2d, wn):
    def body(sp_ref, u_ref, d0_ref, d1_ref, b_ref, w_ref, out_ref):
        dis = lax.rsqrt(d0_ref[:, 0] + d1_ref[:, 0] + 1.0)
        s = jnp.concatenate([sp_ref[0], sp_ref[1]], axis=1)
        uu = jnp.concatenate([u_ref[0], u_ref[1]], axis=1)
        h = jnp.maximum(dis[:, None] * (s + uu) + b_ref[0, :], 0.0)
        v = dis[:, None] * jnp.dot(
            h, w_ref[...], preferred_element_type=jnp.float32)
        out_ref[0, :, :] = v[:, :_HH]
        out_ref[1, :, :] = v[:, _HH:]

    return pl.pallas_call(
        body,
        grid=(_NBLK,),
        in_specs=[
            pl.BlockSpec((_NC, _BLK, _HH), lambda i: (0, i, 0)),
            pl.BlockSpec((_NC, _BLK, _HH), lambda i: (0, i, 0)),
            pl.BlockSpec((_BLK, 16), lambda i: (i, 0)),
            pl.BlockSpec((_BLK, 16), lambda i: (i, 0)),
            pl.BlockSpec((1, _H), lambda i: (0, 0)),
            pl.BlockSpec((_H, _H), lambda i: (0, 0)),
        ],
        out_specs=pl.BlockSpec((_NC, _BLK, _HH), lambda i: (0, i, 0)),
        out_shape=jax.ShapeDtypeStruct((_NC, _N, _HH), jnp.float32),
    )(sp, u, dp0, dp1, b2d, wn)


def _tc_final(sp, u, dp0, dp1, b2d, wl, bl2d, batch3, prot2d, prot_emb,
              a1w, a1b, a2w, a2b, a3w, a3b, a4w, a4b):
    def body(sp_ref, u_ref, d0_ref, d1_ref, b_ref, wl_ref, bl_ref,
             bt_ref, pr_ref, pe_ref, A1_ref, c1_ref, A2_ref, c2_ref, A3_ref,
             c3_ref, A4_ref, c4_ref, out_ref, acc_s, acc_c):
        i = pl.program_id(0)
        dis = lax.rsqrt(d0_ref[:, 0] + d1_ref[:, 0] + 1.0)
        s = jnp.concatenate([sp_ref[0], sp_ref[1]], axis=1)
        uu = jnp.concatenate([u_ref[0], u_ref[1]], axis=1)
        h = jnp.maximum(dis[:, None] * (s + uu) + b_ref[0, :], 0.0)
        y = jnp.dot(h, wl_ref[...], preferred_element_type=jnp.float32) \
            + bl_ref[0, :]
        bb = bt_ref[0, 0, :]
        oh = (lax.broadcasted_iota(jnp.int32, (_B, 1), 0) == bb[None, :]
              ).astype(jnp.float32)
        ps = jnp.dot(oh, y, preferred_element_type=jnp.float32)
        pc = jnp.sum(oh, axis=1, keepdims=True)

        @pl.when(i == 0)
        def _():
            acc_s[...] = ps
            acc_c[...] = jnp.broadcast_to(pc, (_B, _OUT))

        @pl.when(i > 0)
        def _():
            acc_s[...] += ps
            acc_c[...] += jnp.broadcast_to(pc, (_B, _OUT))

        @pl.when(i == _NBLK - 1)
        def _():
            g = acc_s[...] / acc_c[...]
            pr = pr_ref[0, :]
            oh3 = (pr[:, None] == lax.broadcasted_iota(jnp.int32, (1, 3), 1)
                   ).astype(jnp.float32)
            pe = jnp.maximum(
                jnp.dot(oh3, pe_ref[...], preferred_element_type=jnp.float32),
                0.0)
            z = jnp.concatenate(
                [g, pe, jnp.zeros((_B, 6), jnp.float32)], axis=1)
            z = jnp.maximum(
                jnp.dot(z, A1_ref[...], preferred_element_type=jnp.float32)
                + c1_ref[0, :], 0.0)
            z = jnp.maximum(
                jnp.dot(z, A2_ref[...], preferred_element_type=jnp.float32)
                + c2_ref[0, :], 0.0)
            z = jnp.maximum(
                jnp.dot(z, A3_ref[...], preferred_element_type=jnp.float32)
                + c3_ref[0, :], 0.0)
            out_ref[...] = jax.nn.sigmoid(
                jnp.dot(z, A4_ref[...], preferred_element_type=jnp.float32)
                + c4_ref[0, :])

    return pl.pallas_call(
        body,
        grid=(_NBLK,),
        in_specs=[
            pl.BlockSpec((_NC, _BLK, _HH), lambda i: (0, i, 0)),
            pl.BlockSpec((_NC, _BLK, _HH), lambda i: (0, i, 0)),
            pl.BlockSpec((_BLK, 16), lambda i: (i, 0)),
            pl.BlockSpec((_BLK, 16), lambda i: (i, 0)),
            pl.BlockSpec((1, _H), lambda i: (0, 0)),
            pl.BlockSpec((_H, _OUT), lambda i: (0, 0)),
            pl.BlockSpec((1, _OUT), lambda i: (0, 0)),
            pl.BlockSpec((1, 1, _BLK), lambda i: (i, 0, 0)),
            pl.BlockSpec((1, _B), lambda i: (0, 0)),
            pl.BlockSpec((3, 10), lambda i: (0, 0)),
            pl.BlockSpec((144, 128), lambda i: (0, 0)),
            pl.BlockSpec((1, 128), lambda i: (0, 0)),
            pl.BlockSpec((128, 96), lambda i: (0, 0)),
            pl.BlockSpec((1, 96), lambda i: (0, 0)),
            pl.BlockSpec((96, 32), lambda i: (0, 0)),
            pl.BlockSpec((1, 32), lambda i: (0, 0)),
            pl.BlockSpec((32, 1), lambda i: (0, 0)),
            pl.BlockSpec((1, 1), lambda i: (0, 0)),
        ],
        out_specs=pl.BlockSpec((_B, 1), lambda i: (0, 0)),
        out_shape=jax.ShapeDtypeStruct((_B, 1), jnp.float32),
        scratch_shapes=[
            pltpu.VMEM((_B, _OUT), jnp.float32),
            pltpu.VMEM((_B, _OUT), jnp.float32),
        ],
    )(sp, u, dp0, dp1, b2d, wl, bl2d, batch3, prot2d, prot_emb,
      a1w, a1b, a2w, a2b, a3w, a3b, a4w, a4b)


def kernel(x, edge_index, edge_attr, batch, protein, atom_emb, prot_emb,
           W1, b1, W2, b2, W3, b3, W4, b4, Wl, bl, A1, a1, A2, a2, A3, a3,
           A4, a4):
    src = edge_index[0].astype(jnp.int32)
    dst = edge_index[1].astype(jnp.int32)
    ew = edge_attr

    npad = _ER * _CH - _E
    src2 = jnp.pad(src, (0, npad)).reshape(_ER, _CH)
    dst2 = jnp.pad(dst, (0, npad)).reshape(_ER, _CH)
    ew2 = jnp.pad(ew, (0, npad)).reshape(_ER, _CH)

    deg_part = _deg_call(dst2, ew2)
    dp0 = deg_part[0]
    dp1 = deg_part[1]

    u = _tc_pre(x, dp0, dp1, atom_emb, W1[:_D], W1[_D:])

    for (b_cur, w_next) in ((b1, W2), (b2, W3), (b3, W4)):
        sp = _prop_call(u, src2, dst2, ew2)
        u = _tc_mid(sp, u, dp0, dp1, b_cur.reshape(1, _H), w_next)

    sp = _prop_call(u, src2, dst2, ew2)
    out = _tc_final(
        sp, u, dp0, dp1, b4.reshape(1, _H), Wl, bl.reshape(1, _OUT),
        batch.astype(jnp.int32).reshape(_NBLK, 1, _BLK),
        protein.astype(jnp.int32).reshape(1, _B), prot_emb,
        jnp.pad(A1, ((0, 6), (0, 0))), a1.reshape(1, 128),
        A2, a2.reshape(1, 96), A3, a3.reshape(1, 32),
        A4, a4.reshape(1, 1))
    return out
